# baseline (speedup 1.0000x reference)
import jax
import jax.numpy as jnp
from jax import lax
from jax.experimental import pallas as pl
from jax.experimental.pallas import tpu as pltpu
from jax.experimental.pallas import tpu_sc as plsc

BATCH = 16384
DIM = 32
BLOCK = 128
ROWS_PER_BLOCK = BLOCK // DIM
NC = 2
NS = 16
NW = NC * NS
B_PER_W = BATCH // NW
CHUNK = B_PER_W // 2
CGROUPS = CHUNK // 16


def _gmf_body(uidx_hbm, iidx_hbm, user_blk, item_blk, w_hbm, b_hbm,
              out_hbm,
              uidx_v, iidx_v, ublk_v, iblk_v, u_rows, i_rows, out_v,
              w_v, b_v, sem_u, sem_i):
    wid = lax.axis_index("s") * NC + lax.axis_index("c")
    base = wid * B_PER_W

    pltpu.sync_copy(uidx_hbm.at[pl.ds(base, B_PER_W)], uidx_v)
    pltpu.sync_copy(iidx_hbm.at[pl.ds(base, B_PER_W)], iidx_v)
    pltpu.sync_copy(w_hbm, w_v)
    pltpu.sync_copy(b_hbm, b_v)

    def blk(g, carry):
        sl = pl.ds(g * 16, 16)
        ublk_v[sl] = uidx_v[sl] >> 2
        iblk_v[sl] = iidx_v[sl] >> 2
        return carry

    lax.fori_loop(0, B_PER_W // 16, blk, 0)

    lanes = lax.iota(jnp.int32, 16)
    w_lo = w_v[pl.ds(0, 16)]
    w_hi = w_v[pl.ds(16, 16)]
    bias = b_v[...]

    def half(h, carry):
        coff = h * CHUNK
        cu = pltpu.async_copy(
            user_blk.at[ublk_v.at[pl.ds(coff, CHUNK)]], u_rows, sem_u)
        ci = pltpu.async_copy(
            item_blk.at[iblk_v.at[pl.ds(coff, CHUNK)]], i_rows, sem_i)
        cu.wait()
        ci.wait()

        def group(g, carry2):
            gsl = pl.ds(coff + g * 16, 16)
            uq = (uidx_v[gsl] & 3) * DIM
            iq = (iidx_v[gsl] & 3) * DIM
            acc = jnp.zeros((16,), jnp.float32)
            for j in range(16):
                r = g * 16 + j
                us = uq[j]
                js = iq[j]
                ua = u_rows[r, pl.ds(us, 16)]
                ub = u_rows[r, pl.ds(us + 16, 16)]
                ia = i_rows[r, pl.ds(js, 16)]
                ib = i_rows[r, pl.ds(js + 16, 16)]
                p = ua * ia * w_lo + ub * ib * w_hi
                s = jnp.sum(p)
                acc = jnp.where(lanes == j, s, acc)
            logit = acc + bias
            out_v[pl.ds(coff + g * 16, 16)] = 1.0 / (1.0 + jnp.exp(-logit))
            return carry2

        lax.fori_loop(0, CGROUPS, group, 0)
        return carry

    lax.fori_loop(0, 2, half, 0)
    pltpu.sync_copy(out_v, out_hbm.at[pl.ds(base, B_PER_W)])


@jax.jit
def kernel(user_indices, item_indices, user_table, item_table, W, b):
    uidx = user_indices.astype(jnp.int32)
    iidx = item_indices.astype(jnp.int32)
    w32 = W.reshape(DIM).astype(jnp.float32)
    b16 = jnp.broadcast_to(b.astype(jnp.float32), (16,))
    user_blk = user_table.reshape(-1, BLOCK)
    item_blk = item_table.reshape(-1, BLOCK)

    run = pl.kernel(
        _gmf_body,
        out_type=jax.ShapeDtypeStruct((BATCH,), jnp.float32),
        mesh=plsc.VectorSubcoreMesh(core_axis_name="c", subcore_axis_name="s"),
        compiler_params=pltpu.CompilerParams(needs_layout_passes=False),
        scratch_types=[
            pltpu.VMEM((B_PER_W,), jnp.int32),
            pltpu.VMEM((B_PER_W,), jnp.int32),
            pltpu.VMEM((B_PER_W,), jnp.int32),
            pltpu.VMEM((B_PER_W,), jnp.int32),
            pltpu.VMEM((CHUNK, BLOCK), jnp.float32),
            pltpu.VMEM((CHUNK, BLOCK), jnp.float32),
            pltpu.VMEM((B_PER_W,), jnp.float32),
            pltpu.VMEM((DIM,), jnp.float32),
            pltpu.VMEM((16,), jnp.float32),
            pltpu.SemaphoreType.DMA,
            pltpu.SemaphoreType.DMA,
        ],
    )
    out = run(uidx, iidx, user_blk, item_blk, w32, b16)
    return out.reshape(BATCH, 1)
